# baseline (speedup 1.0000x reference)
import jax
import jax.numpy as jnp
from jax.experimental import pallas as pl
from jax.experimental.pallas import tpu as pltpu

_B, _S, _D = 4, 2048, 2048
_MAXLEN = 512
_NS, _SB = 16, 128
_WB = (2 * _D) // _NS
_NB = 8
_PB = _MAXLEN // _NB
_NG = _NS + 1


def _scratch_kernel(pinfo, x_ref, w_ref, emb_ref, b_ref, out_ref,
                    sum_ref, w_vmem):
    g = pl.program_id(0)
    pos = pinfo[0]

    @pl.when(g == 0)
    def _():
        sum_ref[...] = jnp.zeros_like(sum_ref)

    @pl.when(g < _NS)
    def _():
        sum_ref[...] += jnp.sum(x_ref[...], axis=1)
        w_vmem[:, pl.ds(g * _WB, _WB)] = w_ref[...]

    @pl.when(jnp.logical_and(g % 2 == 0, g < _NS))
    def _():
        out_ref[...] = jnp.zeros_like(out_ref)

    @pl.when(g == _NG - 1)
    def _():
        mean = sum_ref[...] * (1.0 / _S)
        ev = emb_ref[0, :, :]
        acc = jax.lax.dot_general(
            mean, w_vmem[:, :_D], (((1,), (1,)), ((), ())),
            preferred_element_type=jnp.float32)
        acc += jax.lax.dot_general(
            ev, w_vmem[:, _D:], (((1,), (1,)), ((), ())),
            preferred_element_type=jnp.float32)
        gate = jax.nn.sigmoid(acc + b_ref[...][None, :])
        val = gate * mean
        out_ref[:, pl.ds(pos % _PB, 1), :] = val[:, None, :]


def _x_map(g, pinfo):
    return (0, jnp.minimum(g, _NS - 1), 0)


def _w_map(g, pinfo):
    return (0, jnp.minimum(g, _NS - 1))


def _emb_map(g, pinfo):
    return (pinfo[0], 0, 0)


def _b_map(g, pinfo):
    return (0,)


def _out_map(g, pinfo):
    pb = pinfo[0] // _PB
    t = jnp.minimum(g // 2, _NB - 1)
    return (0, (pb + 1 + t) % _NB, 0)


_GRID_SPEC = pltpu.PrefetchScalarGridSpec(
    num_scalar_prefetch=1,
    grid=(_NG,),
    in_specs=[
        pl.BlockSpec((_B, _SB, _D), _x_map),
        pl.BlockSpec((_D, _WB), _w_map),
        pl.BlockSpec((1, 1, _D), _emb_map),
        pl.BlockSpec((_D,), _b_map),
    ],
    out_specs=pl.BlockSpec((_B, _PB, _D), _out_map),
    scratch_shapes=[pltpu.VMEM((_B, _D), jnp.float32),
                    pltpu.VMEM((_D, 2 * _D), jnp.float32)],
)


@jax.jit
def _run(current_state, emb_table, W, b, pos):
    pinfo = jnp.reshape(pos, (1,))
    return pl.pallas_call(
        _scratch_kernel,
        grid_spec=_GRID_SPEC,
        out_shape=jax.ShapeDtypeStruct((_B, _MAXLEN, _D), jnp.float32),
        compiler_params=pltpu.CompilerParams(
            dimension_semantics=("arbitrary",)),
    )(pinfo, current_state, W,
      emb_table.reshape(_MAXLEN, 1, _D), b)


def kernel(current_state, emb_table, W, b, step):
    pos = jnp.asarray(step, jnp.int32) % _MAXLEN
    return _run(current_state, emb_table, W, b, pos)


# speedup vs baseline: 1.0706x; 1.0706x over previous
import jax
import jax.numpy as jnp
from jax.experimental import pallas as pl
from jax.experimental.pallas import tpu as pltpu

_B, _S, _D = 4, 2048, 2048
_MAXLEN = 512
_NK, _KB = 8, 256
_NSUB = 2
_SB = _S // _NSUB
_PB = _MAXLEN // _NK
_NG = _NK * _NSUB


def _scratch_kernel(pinfo, x_ref, wa_ref, wb_ref, emb_ref, b_ref, out_ref,
                    mean_ref, acc_ref, psum_ref):
    g = pl.program_id(0)
    k = g // _NSUB
    s = g % _NSUB
    pos = pinfo[0]

    @pl.when(g == 0)
    def _():
        acc_ref[...] = jnp.broadcast_to(b_ref[...][None, :], acc_ref.shape)

    @pl.when(s == 0)
    def _():
        psum_ref[...] = jnp.zeros_like(psum_ref)
        out_ref[...] = jnp.zeros_like(out_ref)

    psum_ref[...] += jnp.sum(x_ref[...], axis=1)

    @pl.when(s == _NSUB - 1)
    def _():
        ms = psum_ref[...] * (1.0 / _S)
        mean_ref[:, pl.ds(k * _KB, _KB)] = ms
        ev = emb_ref[0, :, :]
        acc_ref[...] += jax.lax.dot_general(
            ms, wa_ref[...], (((1,), (1,)), ((), ())),
            preferred_element_type=jnp.float32)
        acc_ref[...] += jax.lax.dot_general(
            ev, wb_ref[...], (((1,), (1,)), ((), ())),
            preferred_element_type=jnp.float32)

        @pl.when(g == _NG - 1)
        def _():
            gate = jax.nn.sigmoid(acc_ref[...])
            val = gate * mean_ref[...]
            out_ref[:, pl.ds(pos % _PB, 1), :] = val[:, None, :]


def _x_map(g, pinfo):
    return (0, g % _NSUB, g // _NSUB)


def _wa_map(g, pinfo):
    return (0, g // _NSUB)


def _wb_map(g, pinfo):
    return (0, _NK + g // _NSUB)


def _emb_map(g, pinfo):
    return (pinfo[0], 0, g // _NSUB)


def _b_map(g, pinfo):
    return (0,)


def _out_map(g, pinfo):
    pb = pinfo[0] // _PB
    return (0, (pb + 1 + g // _NSUB) % _NK, 0)


_GRID_SPEC = pltpu.PrefetchScalarGridSpec(
    num_scalar_prefetch=1,
    grid=(_NG,),
    in_specs=[
        pl.BlockSpec((_B, _SB, _KB), _x_map),
        pl.BlockSpec((_D, _KB), _wa_map),
        pl.BlockSpec((_D, _KB), _wb_map),
        pl.BlockSpec((1, 1, _KB), _emb_map),
        pl.BlockSpec((_D,), _b_map),
    ],
    out_specs=pl.BlockSpec((_B, _PB, _D), _out_map),
    scratch_shapes=[pltpu.VMEM((_B, _D), jnp.float32),
                    pltpu.VMEM((_B, _D), jnp.float32),
                    pltpu.VMEM((_B, _KB), jnp.float32)],
)


@jax.jit
def _run(current_state, emb_table, W, b, pos):
    pinfo = jnp.reshape(pos, (1,))
    return pl.pallas_call(
        _scratch_kernel,
        grid_spec=_GRID_SPEC,
        out_shape=jax.ShapeDtypeStruct((_B, _MAXLEN, _D), jnp.float32),
        compiler_params=pltpu.CompilerParams(
            dimension_semantics=("arbitrary",)),
    )(pinfo, current_state, W, W,
      emb_table.reshape(_MAXLEN, 1, _D), b)


def kernel(current_state, emb_table, W, b, step):
    pos = jnp.asarray(step, jnp.int32) % _MAXLEN
    return _run(current_state, emb_table, W, b, pos)


# speedup vs baseline: 1.1064x; 1.0334x over previous
import jax
import jax.numpy as jnp
from jax.experimental import pallas as pl
from jax.experimental.pallas import tpu as pltpu

_B, _S, _D = 4, 2048, 2048
_MAXLEN = 512
_NK, _KB = 8, 256
_NSUB = 1
_SB = _S // _NSUB
_PB = _MAXLEN // _NK
_NG = _NK * _NSUB


def _scratch_kernel(pinfo, x_ref, wa_ref, wb_ref, emb_ref, b_ref, out_ref,
                    mean_ref, acc_ref, psum_ref):
    g = pl.program_id(0)
    k = g // _NSUB
    s = g % _NSUB
    pos = pinfo[0]

    @pl.when(g == 0)
    def _():
        acc_ref[...] = jnp.broadcast_to(b_ref[...][None, :], acc_ref.shape)

    @pl.when(s == 0)
    def _():
        psum_ref[...] = jnp.zeros_like(psum_ref)
        out_ref[...] = jnp.zeros_like(out_ref)

    psum_ref[...] += jnp.sum(x_ref[...], axis=1)

    @pl.when(s == _NSUB - 1)
    def _():
        ms = psum_ref[...] * (1.0 / _S)
        mean_ref[:, pl.ds(k * _KB, _KB)] = ms
        ev = emb_ref[0, :, :]
        acc_ref[...] += jax.lax.dot_general(
            ms, wa_ref[...], (((1,), (1,)), ((), ())),
            preferred_element_type=jnp.float32)
        acc_ref[...] += jax.lax.dot_general(
            ev, wb_ref[...], (((1,), (1,)), ((), ())),
            preferred_element_type=jnp.float32)

        @pl.when(g == _NG - 1)
        def _():
            gate = jax.nn.sigmoid(acc_ref[...])
            val = gate * mean_ref[...]
            out_ref[:, pl.ds(pos % _PB, 1), :] = val[:, None, :]


def _x_map(g, pinfo):
    return (0, g % _NSUB, g // _NSUB)


def _wa_map(g, pinfo):
    return (0, g // _NSUB)


def _wb_map(g, pinfo):
    return (0, _NK + g // _NSUB)


def _emb_map(g, pinfo):
    return (pinfo[0], 0, g // _NSUB)


def _b_map(g, pinfo):
    return (0,)


def _out_map(g, pinfo):
    pb = pinfo[0] // _PB
    return (0, (pb + 1 + g // _NSUB) % _NK, 0)


_GRID_SPEC = pltpu.PrefetchScalarGridSpec(
    num_scalar_prefetch=1,
    grid=(_NG,),
    in_specs=[
        pl.BlockSpec((_B, _SB, _KB), _x_map),
        pl.BlockSpec((_D, _KB), _wa_map),
        pl.BlockSpec((_D, _KB), _wb_map),
        pl.BlockSpec((1, 1, _KB), _emb_map),
        pl.BlockSpec((_D,), _b_map),
    ],
    out_specs=pl.BlockSpec((_B, _PB, _D), _out_map),
    scratch_shapes=[pltpu.VMEM((_B, _D), jnp.float32),
                    pltpu.VMEM((_B, _D), jnp.float32),
                    pltpu.VMEM((_B, _KB), jnp.float32)],
)


@jax.jit
def _run(current_state, emb_table, W, b, pos):
    pinfo = jnp.reshape(pos, (1,))
    return pl.pallas_call(
        _scratch_kernel,
        grid_spec=_GRID_SPEC,
        out_shape=jax.ShapeDtypeStruct((_B, _MAXLEN, _D), jnp.float32),
        compiler_params=pltpu.CompilerParams(
            dimension_semantics=("arbitrary",)),
    )(pinfo, current_state, W, W,
      emb_table.reshape(_MAXLEN, 1, _D), b)


def kernel(current_state, emb_table, W, b, step):
    pos = jnp.asarray(step, jnp.int32) % _MAXLEN
    return _run(current_state, emb_table, W, b, pos)
